# baseline (speedup 1.0000x reference)
import functools

import jax
import jax.numpy as jnp
from jax import lax
from jax.experimental import pallas as pl
from jax.experimental.pallas import tpu as pltpu
from jax.experimental.pallas import tpu_sc as plsc

_COL = 777
_NCOLS = 1000
_B = 16384


def kernel(truth_values):
    info = plsc.get_sparse_core_info()
    num_workers = info.num_cores * info.num_subcores
    rows_per_worker = _B // num_workers
    num_lanes = info.num_lanes

    mesh = plsc.VectorSubcoreMesh(core_axis_name="c", subcore_axis_name="s")

    @functools.partial(
        pl.kernel,
        mesh=mesh,
        out_type=jax.ShapeDtypeStruct((_B,), jnp.float32),
        scratch_types=[
            pltpu.VMEM((rows_per_worker,), jnp.int32),
            pltpu.VMEM((rows_per_worker,), jnp.float32),
            pltpu.SemaphoreType.DMA,
        ],
    )
    def column_select(tv_hbm, out_hbm, idx_v, vals_v, sem):
        wid = lax.axis_index("s") * info.num_cores + lax.axis_index("c")
        base = wid * rows_per_worker
        lane_iota = lax.iota(jnp.int32, num_lanes)
        for j in range(rows_per_worker // num_lanes):
            rows = base + j * num_lanes + lane_iota
            idx_v[pl.ds(j * num_lanes, num_lanes)] = rows * _NCOLS + _COL
        pltpu.async_copy(tv_hbm.at[idx_v], vals_v, sem).wait()
        pltpu.sync_copy(vals_v, out_hbm.at[pl.ds(base, rows_per_worker)])

    flat = truth_values.reshape(-1)
    return column_select(flat).reshape(_B, 1)


# speedup vs baseline: 1.9660x; 1.9660x over previous
import jax
import jax.numpy as jnp
from jax.experimental import pallas as pl

_COL = 777
_B = 16384
_LANE_BLOCK = _COL // 128
_LANE_IN_BLOCK = _COL % 128
_ROWS_PER_STEP = 2048


def _extract_body(tv_ref, out_ref):
    out_ref[...] = tv_ref[:, _LANE_IN_BLOCK:_LANE_IN_BLOCK + 1]


def kernel(truth_values):
    grid = (_B // _ROWS_PER_STEP,)
    return pl.pallas_call(
        _extract_body,
        grid=grid,
        in_specs=[
            pl.BlockSpec((_ROWS_PER_STEP, 128), lambda i: (i, _LANE_BLOCK)),
        ],
        out_specs=pl.BlockSpec((_ROWS_PER_STEP, 1), lambda i: (i, 0)),
        out_shape=jax.ShapeDtypeStruct((_B, 1), jnp.float32),
    )(truth_values)


# speedup vs baseline: 2.2094x; 1.1238x over previous
import jax
import jax.numpy as jnp
from jax.experimental import pallas as pl

_COL = 777
_B = 16384
_LANE_BLOCK = _COL // 128
_LANE_IN_BLOCK = _COL % 128
_ROWS_PER_STEP = 2048


def _extract_body(tv_ref, out_ref):
    col = tv_ref[:, _LANE_IN_BLOCK]
    out_ref[...] = col.reshape(_ROWS_PER_STEP // 128, 128)


def kernel(truth_values):
    grid = (_B // _ROWS_PER_STEP,)
    packed = pl.pallas_call(
        _extract_body,
        grid=grid,
        in_specs=[
            pl.BlockSpec((_ROWS_PER_STEP, 128), lambda i: (i, _LANE_BLOCK)),
        ],
        out_specs=pl.BlockSpec((_ROWS_PER_STEP // 128, 128), lambda i: (i, 0)),
        out_shape=jax.ShapeDtypeStruct((_B // 128, 128), jnp.float32),
    )(truth_values)
    return packed.reshape(_B, 1)


# speedup vs baseline: 30.7665x; 13.9254x over previous
import jax
import jax.numpy as jnp
from jax.experimental import pallas as pl

_COL = 777
_B = 16384
_ROW_TILE = _COL // 8
_SUBLANE = _COL % 8
_COLS_PER_STEP = 2048


def _extract_body(tvT_ref, out_ref):
    strip = tvT_ref[_SUBLANE, :]
    out_ref[...] = strip.reshape(_COLS_PER_STEP // 128, 128)


def kernel(truth_values):
    tvT = truth_values.T
    grid = (_B // _COLS_PER_STEP,)
    packed = pl.pallas_call(
        _extract_body,
        grid=grid,
        in_specs=[
            pl.BlockSpec((8, _COLS_PER_STEP), lambda i: (_ROW_TILE, i)),
        ],
        out_specs=pl.BlockSpec((_COLS_PER_STEP // 128, 128), lambda i: (i, 0)),
        out_shape=jax.ShapeDtypeStruct((_B // 128, 128), jnp.float32),
    )(tvT)
    return packed.reshape(_B, 1)


# speedup vs baseline: 48.8304x; 1.5871x over previous
import jax
import jax.numpy as jnp
from jax.experimental import pallas as pl
from jax.experimental.pallas import tpu as pltpu

_COL = 777
_B = 16384


def _copy_body(tvT_ref, out_ref, sem):
    pltpu.make_async_copy(tvT_ref.at[_COL], out_ref, sem).start()
    pltpu.make_async_copy(tvT_ref.at[_COL], out_ref, sem).wait()


def kernel(truth_values):
    tvT = truth_values.T
    flat = pl.pallas_call(
        _copy_body,
        in_specs=[pl.BlockSpec(memory_space=pl.ANY)],
        out_specs=pl.BlockSpec(memory_space=pl.ANY),
        out_shape=jax.ShapeDtypeStruct((_B,), jnp.float32),
        scratch_shapes=[pltpu.SemaphoreType.DMA],
    )(tvT)
    return flat.reshape(_B, 1)
